# baseline (speedup 1.0000x reference)
import functools

import jax
import jax.numpy as jnp
from jax.experimental import pallas as pl
from jax.experimental.pallas import tpu as pltpu

_NCHUNK = 8
_DEPTH = 4


def _mlp_kernel(x_hbm, w1_ref, b1_ref, w2_ref, b2_ref, w3_ref, b3_ref,
                o_ref, xbuf, sems):
    steps, batch = x_hbm.shape
    ch = batch // _NCHUNK

    def copy_in(i, slot):
        return pltpu.make_async_copy(
            x_hbm.at[:, pl.ds(i * ch, ch)], xbuf.at[slot], sems.at[slot])

    for i in range(_DEPTH - 1):
        copy_in(i, i).start()

    w1 = w1_ref[...]
    b1 = b1_ref[...].T
    w2 = w2_ref[...]
    b2 = b2_ref[...].T
    w3 = w3_ref[...].T
    b3 = b3_ref[0, 0]

    for i in range(_NCHUNK):
        slot = i % _DEPTH
        copy_in(i, slot).wait()
        x = xbuf[slot]
        h = jax.lax.dot_general(
            w1, x, (((1,), (0,)), ((), ())),
            preferred_element_type=jnp.float32)
        h = jnp.maximum(h + b1, 0.0)
        z = jax.lax.dot_general(
            w2, h, (((1,), (0,)), ((), ())),
            preferred_element_type=jnp.float32)
        h2 = jnp.maximum(z + b2, 0.0) * w3
        o_ref[pl.ds(i * ch, ch)] = jnp.sum(h2, axis=0) + b3
        nxt = i + _DEPTH - 1
        if nxt < _NCHUNK:
            copy_in(nxt, nxt % _DEPTH).start()


@functools.partial(jax.jit, static_argnames=())
def kernel(modes_vec, W1, b1, W2, b2, W3, b3, type_n):
    del type_n
    batch, steps = modes_vec.shape

    xt = modes_vec.T

    full = lambda: (0, 0)
    out = pl.pallas_call(
        _mlp_kernel,
        in_specs=[
            pl.BlockSpec(memory_space=pltpu.MemorySpace.HBM),
            pl.BlockSpec(W1.shape, full),
            pl.BlockSpec((1, W1.shape[0]), full),
            pl.BlockSpec(W2.shape, full),
            pl.BlockSpec((1, W2.shape[0]), full),
            pl.BlockSpec(W3.shape, full),
            pl.BlockSpec((1, 1), full),
        ],
        out_specs=pl.BlockSpec(memory_space=pltpu.MemorySpace.VMEM),
        out_shape=jax.ShapeDtypeStruct((batch,), jnp.float32),
        scratch_shapes=[
            pltpu.VMEM((_DEPTH, steps, batch // _NCHUNK), jnp.float32),
            pltpu.SemaphoreType.DMA((_DEPTH,)),
        ],
    )(xt, W1, b1.reshape(1, -1), W2, b2.reshape(1, -1), W3, b3.reshape(1, -1))
    return out.reshape(batch, 1)


# speedup vs baseline: 1.0527x; 1.0527x over previous
import functools

import jax
import jax.numpy as jnp
from jax.experimental import pallas as pl
from jax.experimental.pallas import tpu as pltpu

_NCHUNK = 8
_DEPTH = 8


def _mlp_kernel(x_hbm, w1_ref, b1_ref, w2_ref, b2_ref, w3_ref, b3_ref,
                o_ref, xbuf, sems):
    steps, batch = x_hbm.shape
    ch = batch // _NCHUNK

    def copy_in(i, slot):
        return pltpu.make_async_copy(
            x_hbm.at[:, pl.ds(i * ch, ch)], xbuf.at[slot], sems.at[slot])

    for i in range(_NCHUNK):
        copy_in(i, i % _DEPTH).start()

    w1 = w1_ref[...]
    b1 = b1_ref[...].T
    w2 = w2_ref[...]
    b2 = b2_ref[...].T
    w3 = w3_ref[...].T
    b3 = b3_ref[0, 0]

    for i in range(_NCHUNK):
        slot = i % _DEPTH
        copy_in(i, slot).wait()
        x = xbuf[slot]
        h = jax.lax.dot_general(
            w1, x, (((1,), (0,)), ((), ())),
            preferred_element_type=jnp.float32)
        h = jnp.maximum(h + b1, 0.0)
        z = jax.lax.dot_general(
            w2, h, (((1,), (0,)), ((), ())),
            preferred_element_type=jnp.float32)
        h2 = jnp.maximum(z + b2, 0.0) * w3
        o_ref[pl.ds(i * ch, ch)] = jnp.sum(h2, axis=0) + b3


@functools.partial(jax.jit, static_argnames=())
def kernel(modes_vec, W1, b1, W2, b2, W3, b3, type_n):
    del type_n
    batch, steps = modes_vec.shape

    xt = modes_vec.T

    full = lambda: (0, 0)
    out = pl.pallas_call(
        _mlp_kernel,
        in_specs=[
            pl.BlockSpec(memory_space=pltpu.MemorySpace.HBM),
            pl.BlockSpec(W1.shape, full),
            pl.BlockSpec((1, W1.shape[0]), full),
            pl.BlockSpec(W2.shape, full),
            pl.BlockSpec((1, W2.shape[0]), full),
            pl.BlockSpec(W3.shape, full),
            pl.BlockSpec((1, 1), full),
        ],
        out_specs=pl.BlockSpec(memory_space=pltpu.MemorySpace.VMEM),
        out_shape=jax.ShapeDtypeStruct((batch,), jnp.float32),
        scratch_shapes=[
            pltpu.VMEM((_DEPTH, steps, batch // _NCHUNK), jnp.float32),
            pltpu.SemaphoreType.DMA((_DEPTH,)),
        ],
    )(xt, W1, b1.reshape(1, -1), W2, b2.reshape(1, -1), W3, b3.reshape(1, -1))
    return out.reshape(batch, 1)


# speedup vs baseline: 1.1529x; 1.0953x over previous
import functools

import jax
import jax.numpy as jnp
from jax.experimental import pallas as pl
from jax.experimental.pallas import tpu as pltpu

_LANE_BLK = 8192


def _mlp_kernel(x_ref, w1_ref, b1_ref, w2_ref, b2_ref, w3_ref, b3_ref, o_ref):
    x = x_ref[...]
    h = jax.lax.dot_general(
        w1_ref[...], x, (((1,), (0,)), ((), ())),
        preferred_element_type=jnp.float32)
    h = jnp.maximum(h + b1_ref[...].T, 0.0)
    z = jax.lax.dot_general(
        w2_ref[...], h, (((1,), (0,)), ((), ())),
        preferred_element_type=jnp.float32)
    h2 = jnp.maximum(z + b2_ref[...].T, 0.0) * w3_ref[...].T
    o_ref[...] = jnp.sum(h2, axis=0) + b3_ref[0, 0]


@functools.partial(jax.jit, static_argnames=())
def kernel(modes_vec, W1, b1, W2, b2, W3, b3, type_n):
    del type_n
    batch, steps = modes_vec.shape
    blk = min(_LANE_BLK, batch)
    grid = (batch // blk,)

    xt = modes_vec.T

    full = lambda i: (0, 0)
    outt = pl.pallas_call(
        _mlp_kernel,
        grid=grid,
        in_specs=[
            pl.BlockSpec((steps, blk), lambda i: (0, i)),
            pl.BlockSpec(W1.shape, full),
            pl.BlockSpec((1, W1.shape[0]), full),
            pl.BlockSpec(W2.shape, full),
            pl.BlockSpec((1, W2.shape[0]), full),
            pl.BlockSpec(W3.shape, full),
            pl.BlockSpec((1, 1), full),
        ],
        out_specs=pl.BlockSpec((blk,), lambda i: (i,)),
        out_shape=jax.ShapeDtypeStruct((batch,), jnp.float32),
        compiler_params=pltpu.CompilerParams(
            dimension_semantics=("parallel",),
        ),
    )(xt, W1, b1.reshape(1, -1), W2, b2.reshape(1, -1), W3, b3.reshape(1, -1))
    return outt.reshape(batch, 1)
